# baseline (speedup 1.0000x reference)
import jax
import jax.numpy as jnp
from jax import lax
from jax.experimental import pallas as pl
from jax.experimental.pallas import tpu as pltpu
from jax.experimental.pallas import tpu_sc as plsc

N_NODES = 10000
N_EDGES = 320000
D = 128
D_OUT = 64

NC = 2
NS = 16
CH = 128
NCH = 160
E_PAD = NS * NCH * CH

ACC_ROWS = 10112
ROWS_PER_TILE = ACC_ROWS // NS

NBUF = 2
STAGE = 40

_MESH = plsc.VectorSubcoreMesh(core_axis_name="c", subcore_axis_name="s",
                               num_cores=NC, num_subcores=NS)

_SCRATCH = [
    pltpu.VMEM((STAGE, CH), jnp.int32),
    pltpu.VMEM((STAGE, CH), jnp.int32),
    [pltpu.VMEM((CH, D), jnp.float32)] * NBUF,
    [pltpu.SemaphoreType.DMA] * NBUF,
    [pltpu.SemaphoreType.DMA] * NBUF,
    pltpu.VMEM_SHARED((ACC_ROWS, D), jnp.float32),
]


def _zero_acc(zrow_hbm, acc, s):
    r0 = s * ROWS_PER_TILE
    pltpu.sync_copy(zrow_hbm.at[pl.ds(r0, ROWS_PER_TILE)],
                    acc.at[pl.ds(r0, ROWS_PER_TILE)])
    return r0


def _gather_scatter(h_hbm, src_hbm, dst_hbm, sidx, didx, rows, gsem, ssem,
                    acc, s):
    def group(g, carry):
        j0 = g * NBUF
        gathers = [
            pltpu.async_copy(h_hbm.at[sidx.at[j0 + k]], rows[k], gsem[k])
            for k in range(NBUF)
        ]
        scatters = []
        for k in range(NBUF):
            gathers[k].wait()
            scatters.append(
                pltpu.async_copy(rows[k], acc.at[didx.at[j0 + k]], ssem[k],
                                 add=True))
        for sc in scatters:
            sc.wait()
        return carry

    for p0 in range(0, NCH, STAGE):
        pltpu.sync_copy(src_hbm.at[s, pl.ds(p0, STAGE)], sidx)
        pltpu.sync_copy(dst_hbm.at[s, pl.ds(p0, STAGE)], didx)
        lax.fori_loop(0, STAGE // NBUF, group, 0)


def _count_scatter(ones_hbm, dst_hbm, didx, rows, ssem, acc, s):
    pltpu.sync_copy(ones_hbm, rows[0])

    def group(g, carry):
        j0 = g * NBUF
        scatters = [
            pltpu.async_copy(rows[0], acc.at[didx.at[j0 + k]], ssem[k],
                             add=True)
            for k in range(NBUF)
        ]
        for sc in scatters:
            sc.wait()
        return carry

    for p0 in range(0, NCH, STAGE):
        pltpu.sync_copy(dst_hbm.at[s, pl.ds(p0, STAGE)], didx)
        lax.fori_loop(0, STAGE // NBUF, group, 0)


def _writeout(acc, out_hbm, r0):
    pltpu.sync_copy(acc.at[pl.ds(r0, ROWS_PER_TILE)],
                    out_hbm.at[pl.ds(r0, ROWS_PER_TILE)])


def _agg0_body(h_hbm, src_hbm, dst_hbm, ones_hbm, zrow_hbm,
               part_hbm, cnt_hbm,
               sidx, didx, rows, gsem, ssem, acc):
    c = lax.axis_index("c")
    s = lax.axis_index("s")
    r0 = _zero_acc(zrow_hbm, acc, s)
    plsc.subcore_barrier()

    @pl.when(c == 0)
    def _():
        _gather_scatter(h_hbm, src_hbm, dst_hbm, sidx, didx, rows, gsem,
                        ssem, acc, s)

    @pl.when(c == 1)
    def _():
        _count_scatter(ones_hbm, dst_hbm, didx, rows, ssem, acc, s)

    plsc.subcore_barrier()

    @pl.when(c == 0)
    def _():
        _writeout(acc, part_hbm, r0)

    @pl.when(c == 1)
    def _():
        _writeout(acc, cnt_hbm, r0)


_agg0 = pl.kernel(
    _agg0_body,
    out_type=(jax.ShapeDtypeStruct((ACC_ROWS, D), jnp.float32),
              jax.ShapeDtypeStruct((ACC_ROWS, D), jnp.float32)),
    mesh=_MESH,
    scratch_types=_SCRATCH,
)


def _agg_body(h_hbm, src_hbm, dst_hbm, zrow_hbm, part_hbm,
              sidx, didx, rows, gsem, ssem, acc):
    c = lax.axis_index("c")
    s = lax.axis_index("s")

    @pl.when(c == 0)
    def _():
        r0 = _zero_acc(zrow_hbm, acc, s)
        plsc.subcore_barrier()
        _gather_scatter(h_hbm, src_hbm, dst_hbm, sidx, didx, rows, gsem,
                        ssem, acc, s)
        plsc.subcore_barrier()
        _writeout(acc, part_hbm, r0)


_agg = pl.kernel(
    _agg_body,
    out_type=jax.ShapeDtypeStruct((ACC_ROWS, D), jnp.float32),
    mesh=_MESH,
    scratch_types=_SCRATCH,
)


BLK = 2000


def _dense_blockspecs():
    return [
        pl.BlockSpec((BLK, D), lambda i: (i, 0)),
        pl.BlockSpec((BLK, D), lambda i: (i, 0)),
        pl.BlockSpec((BLK, D), lambda i: (i, 0)),
        pl.BlockSpec((D, D), lambda i: (0, 0)),
        pl.BlockSpec((D, D), lambda i: (0, 0)),
        pl.BlockSpec((1, D), lambda i: (0, 0)),
    ]


def _mean_combine(a_ref, c_ref):
    cnt = jnp.maximum(c_ref[:, :1], 1.0)
    return a_ref[...] / cnt


def _dot(a, b):
    return jnp.dot(a, b, preferred_element_type=jnp.float32,
                   precision=lax.Precision.HIGHEST)


def _dense_kernel(h_ref, a_ref, c_ref, wl_ref, wr_ref, b_ref, o_ref):
    mean = _mean_combine(a_ref, c_ref)
    out = _dot(mean, wl_ref[...]) + _dot(h_ref[...], wr_ref[...]) + b_ref[...]
    o_ref[...] = jnp.maximum(out, 0.0)


def _final_kernel(h_ref, a_ref, c_ref, wl_ref, wr_ref, b_ref, w_ref, bo_ref,
                  o_ref):
    mean = _mean_combine(a_ref, c_ref)
    h3 = _dot(mean, wl_ref[...]) + _dot(h_ref[...], wr_ref[...]) + b_ref[...]
    h3 = jnp.maximum(h3, 0.0)
    logits = _dot(h3, w_ref[...]) + bo_ref[...]
    m = jnp.max(logits, axis=1, keepdims=True)
    shifted = logits - m
    lse = jnp.log(jnp.sum(jnp.exp(shifted), axis=1, keepdims=True))
    o_ref[...] = shifted - lse


def _dense(h, part, cnt, Wl, Wr, b):
    return pl.pallas_call(
        _dense_kernel,
        grid=(N_NODES // BLK,),
        in_specs=_dense_blockspecs(),
        out_specs=pl.BlockSpec((BLK, D), lambda i: (i, 0)),
        out_shape=jax.ShapeDtypeStruct((N_NODES, D), jnp.float32),
    )(h, part, cnt, Wl, Wr, b.reshape(1, D))


def _final(h, part, cnt, Wl, Wr, b, W, bo):
    specs = _dense_blockspecs() + [
        pl.BlockSpec((D, D_OUT), lambda i: (0, 0)),
        pl.BlockSpec((1, D_OUT), lambda i: (0, 0)),
    ]
    return pl.pallas_call(
        _final_kernel,
        grid=(N_NODES // BLK,),
        in_specs=specs,
        out_specs=pl.BlockSpec((BLK, D_OUT), lambda i: (i, 0)),
        out_shape=jax.ShapeDtypeStruct((N_NODES, D_OUT), jnp.float32),
    )(h, part, cnt, Wl, Wr, b.reshape(1, D), W, bo.reshape(1, D_OUT))


def kernel(x, edge_index, Wl0, Wr0, b0, Wl1, Wr1, b1, Wl2, Wr2, b2, W, b):
    src = edge_index[0].astype(jnp.int32)
    dst = edge_index[1].astype(jnp.int32)
    pad = E_PAD - N_EDGES
    src3 = jnp.concatenate([src, jnp.zeros((pad,), jnp.int32)]).reshape(
        NS, NCH, CH)
    dst3 = jnp.concatenate([dst, jnp.full((pad,), N_NODES, jnp.int32)]
                           ).reshape(NS, NCH, CH)

    zrow = jnp.zeros((ACC_ROWS, D), jnp.float32)
    ones = jnp.ones((CH, D), jnp.float32)

    part0, cnt = _agg0(x, src3, dst3, ones, zrow)
    h1 = _dense(x, part0, cnt, Wl0, Wr0, b0)
    part1 = _agg(h1, src3, dst3, zrow)
    h2 = _dense(h1, part1, cnt, Wl1, Wr1, b1)
    part2 = _agg(h2, src3, dst3, zrow)
    return _final(h2, part2, cnt, Wl2, Wr2, b2, W, b)


# speedup vs baseline: 1.2529x; 1.2529x over previous
import functools

import jax
import jax.numpy as jnp
from jax import lax
from jax.experimental import pallas as pl
from jax.experimental.pallas import tpu as pltpu
from jax.experimental.pallas import tpu_sc as plsc

N_NODES = 10000
N_EDGES = 320000
D = 128
D_OUT = 64

NC = 2
NS = 16
CH = 128
NCH = 80
E_PAD = NC * NS * NCH * CH

ACC_ROWS = 10112
ROWS_PER_TILE = ACC_ROWS // NS

_MESH = plsc.VectorSubcoreMesh(core_axis_name="c", subcore_axis_name="s",
                               num_cores=NC, num_subcores=NS)


NBUF = 2
NCH0 = 120
NCH1 = 40
STAGE = 40
E_SPLIT = NS * NCH0 * CH

_AGG_KERNEL_ARGS = dict(
    out_type=jax.ShapeDtypeStruct((NC, ACC_ROWS, D), jnp.float32),
    mesh=_MESH,
    scratch_types=[
        pltpu.VMEM((STAGE, CH), jnp.int32),
        pltpu.VMEM((STAGE, CH), jnp.int32),
        [pltpu.VMEM((CH, D), jnp.float32)] * NBUF,
        [pltpu.SemaphoreType.DMA] * NBUF,
        [pltpu.SemaphoreType.DMA] * NBUF,
        pltpu.VMEM_SHARED((ACC_ROWS, D), jnp.float32),
    ],
)


def _agg_body(h_hbm, srca_hbm, dsta_hbm, srcb_hbm, dstb_hbm, zrow_hbm,
              part_hbm, sidx, didx, rows, gsem, ssem, acc):
    c = lax.axis_index("c")
    s = lax.axis_index("s")

    r0 = s * ROWS_PER_TILE
    pltpu.sync_copy(zrow_hbm.at[pl.ds(r0, ROWS_PER_TILE)],
                    acc.at[pl.ds(r0, ROWS_PER_TILE)])
    plsc.subcore_barrier()

    def group(g, carry):
        j0 = g * NBUF
        gathers = [
            pltpu.async_copy(h_hbm.at[sidx.at[j0 + k]], rows[k], gsem[k])
            for k in range(NBUF)
        ]
        scatters = []
        for k in range(NBUF):
            gathers[k].wait()
            scatters.append(
                pltpu.async_copy(rows[k], acc.at[didx.at[j0 + k]], ssem[k],
                                 add=True))
        for sc in scatters:
            sc.wait()
        return carry

    def run(src_hbm, dst_hbm, nch):
        for p0 in range(0, nch, STAGE):
            n = min(STAGE, nch - p0)
            pltpu.sync_copy(src_hbm.at[s, pl.ds(p0, n)],
                            sidx.at[pl.ds(0, n)])
            pltpu.sync_copy(dst_hbm.at[s, pl.ds(p0, n)],
                            didx.at[pl.ds(0, n)])
            lax.fori_loop(0, n // NBUF, group, 0)

    @pl.when(c == 0)
    def _():
        run(srca_hbm, dsta_hbm, NCH0)

    @pl.when(c == 1)
    def _():
        run(srcb_hbm, dstb_hbm, NCH1)

    plsc.subcore_barrier()

    pltpu.sync_copy(acc.at[pl.ds(r0, ROWS_PER_TILE)],
                    part_hbm.at[c, pl.ds(r0, ROWS_PER_TILE)])


_agg = pl.kernel(_agg_body, **_AGG_KERNEL_ARGS)


CBUF = 4

_CNT_KERNEL_ARGS = dict(
    out_type=jax.ShapeDtypeStruct((NC, ACC_ROWS, D), jnp.float32),
    mesh=_MESH,
    scratch_types=[
        pltpu.VMEM((NCH, CH), jnp.int32),
        pltpu.VMEM((CH, D), jnp.float32),
        [pltpu.SemaphoreType.DMA] * CBUF,
        pltpu.VMEM_SHARED((ACC_ROWS, D), jnp.float32),
    ],
)


def _counts_body(ones_hbm, dst_hbm, zrow_hbm, cntp_hbm, didx, ones_v, ssem,
                 cacc):
    c = lax.axis_index("c")
    s = lax.axis_index("s")
    pltpu.sync_copy(dst_hbm.at[c, s], didx)
    r0 = s * ROWS_PER_TILE
    pltpu.sync_copy(zrow_hbm.at[pl.ds(r0, ROWS_PER_TILE)],
                    cacc.at[pl.ds(r0, ROWS_PER_TILE)])
    pltpu.sync_copy(ones_hbm, ones_v)
    plsc.subcore_barrier()

    def group(g, carry):
        j0 = g * CBUF
        scatters = [
            pltpu.async_copy(ones_v, cacc.at[didx.at[j0 + k]], ssem[k],
                             add=True)
            for k in range(CBUF)
        ]
        for sc in scatters:
            sc.wait()
        return carry

    lax.fori_loop(0, NCH // CBUF, group, 0)
    plsc.subcore_barrier()
    pltpu.sync_copy(cacc.at[pl.ds(r0, ROWS_PER_TILE)],
                    cntp_hbm.at[c, pl.ds(r0, ROWS_PER_TILE)])


_counts = pl.kernel(_counts_body, **_CNT_KERNEL_ARGS)


BLK = 2000


def _dense_blockspecs():
    return [
        pl.BlockSpec((BLK, D), lambda i: (i, 0)),
        pl.BlockSpec((1, BLK, D), lambda i: (0, i, 0)),
        pl.BlockSpec((1, BLK, D), lambda i: (1, i, 0)),
        pl.BlockSpec((1, BLK, D), lambda i: (0, i, 0)),
        pl.BlockSpec((1, BLK, D), lambda i: (1, i, 0)),
        pl.BlockSpec((D, D), lambda i: (0, 0)),
        pl.BlockSpec((D, D), lambda i: (0, 0)),
        pl.BlockSpec((1, D), lambda i: (0, 0)),
    ]


def _mean_combine(a0_ref, a1_ref, c0_ref, c1_ref):
    cnt = c0_ref[0, :, :1] + c1_ref[0, :, :1]
    cnt = jnp.maximum(cnt, 1.0)
    return (a0_ref[0] + a1_ref[0]) / cnt


def _dot(a, b):
    return jnp.dot(a, b, preferred_element_type=jnp.float32,
                   precision=lax.Precision.HIGHEST)


def _dense_kernel(h_ref, a0_ref, a1_ref, c0_ref, c1_ref, wl_ref, wr_ref,
                  b_ref, o_ref):
    mean = _mean_combine(a0_ref, a1_ref, c0_ref, c1_ref)
    out = _dot(mean, wl_ref[...]) + _dot(h_ref[...], wr_ref[...]) + b_ref[...]
    o_ref[...] = jnp.maximum(out, 0.0)


def _final_kernel(h_ref, a0_ref, a1_ref, c0_ref, c1_ref, wl_ref, wr_ref,
                  b_ref, w_ref, bo_ref, o_ref):
    mean = _mean_combine(a0_ref, a1_ref, c0_ref, c1_ref)
    h3 = _dot(mean, wl_ref[...]) + _dot(h_ref[...], wr_ref[...]) + b_ref[...]
    h3 = jnp.maximum(h3, 0.0)
    logits = _dot(h3, w_ref[...]) + bo_ref[...]
    m = jnp.max(logits, axis=1, keepdims=True)
    shifted = logits - m
    lse = jnp.log(jnp.sum(jnp.exp(shifted), axis=1, keepdims=True))
    o_ref[...] = shifted - lse


def _dense(h, part, cntp, Wl, Wr, b):
    return pl.pallas_call(
        _dense_kernel,
        grid=(N_NODES // BLK,),
        in_specs=_dense_blockspecs(),
        out_specs=pl.BlockSpec((BLK, D), lambda i: (i, 0)),
        out_shape=jax.ShapeDtypeStruct((N_NODES, D), jnp.float32),
    )(h, part, part, cntp, cntp, Wl, Wr, b.reshape(1, D))


def _final(h, part, cntp, Wl, Wr, b, W, bo):
    specs = _dense_blockspecs() + [
        pl.BlockSpec((D, D_OUT), lambda i: (0, 0)),
        pl.BlockSpec((1, D_OUT), lambda i: (0, 0)),
    ]
    return pl.pallas_call(
        _final_kernel,
        grid=(N_NODES // BLK,),
        in_specs=specs,
        out_specs=pl.BlockSpec((BLK, D_OUT), lambda i: (i, 0)),
        out_shape=jax.ShapeDtypeStruct((N_NODES, D_OUT), jnp.float32),
    )(h, part, part, cntp, cntp, Wl, Wr, b.reshape(1, D), W,
      bo.reshape(1, D_OUT))


def kernel(x, edge_index, Wl0, Wr0, b0, Wl1, Wr1, b1, Wl2, Wr2, b2, W, b):
    src = edge_index[0].astype(jnp.int32)
    dst = edge_index[1].astype(jnp.int32)
    pad = E_PAD - N_EDGES
    src_p = jnp.concatenate([src, jnp.zeros((pad,), jnp.int32)])
    dst_p = jnp.concatenate([dst, jnp.full((pad,), N_NODES, jnp.int32)])
    src4 = src_p.reshape(NC, NS, NCH, CH)
    dst4 = dst_p.reshape(NC, NS, NCH, CH)
    srca = src_p[:E_SPLIT].reshape(NS, NCH0, CH)
    dsta = dst_p[:E_SPLIT].reshape(NS, NCH0, CH)
    srcb = src_p[E_SPLIT:].reshape(NS, NCH1, CH)
    dstb = dst_p[E_SPLIT:].reshape(NS, NCH1, CH)

    zrow = jnp.zeros((ACC_ROWS, D), jnp.float32)
    ones = jnp.ones((CH, D), jnp.float32)
    cntp = _counts(ones, dst4, zrow)
    part0 = _agg(x, srca, dsta, srcb, dstb, zrow)
    h1 = _dense(x, part0, cntp, Wl0, Wr0, b0)
    part1 = _agg(h1, srca, dsta, srcb, dstb, zrow)
    h2 = _dense(h1, part1, cntp, Wl1, Wr1, b1)
    part2 = _agg(h2, srca, dsta, srcb, dstb, zrow)
    return _final(h2, part2, cntp, Wl2, Wr2, b2, W, b)
